# Initial kernel scaffold; baseline (speedup 1.0000x reference)
import jax
import jax.numpy as jnp
from jax.experimental import pallas as pl


def kernel(features, bbox_W, bbox_b, conf_W, conf_b):
    raise NotImplementedError("write your pallas kernel here")


# speedup vs baseline: 2.2629x; 2.2629x over previous
import functools

import numpy as np
import jax
import jax.numpy as jnp
from jax import lax
from jax.experimental import pallas as pl
from jax.experimental.pallas import tpu as pltpu

NUM_ANCHORS = 16
K = 10
FEATURE_DIM = 128
ROW_TILE = 1280


def _head_kernel(x_ref, wb_ref, bb_ref, cw_ref, cb_ref, e_ref,
                 det_ref, vals_ref, mask_ref):
    m_rows = x_ref.shape[0]
    x = x_ref[:]
    bbox = jnp.dot(x, wb_ref[:], preferred_element_type=jnp.float32) + bb_ref[:]
    logits_t = lax.dot_general(cw_ref[:], x, (((0,), (1,)), ((), ())),
                               preferred_element_type=jnp.float32) + cb_ref[:]
    c = jax.nn.sigmoid(logits_t)

    iota_a = lax.broadcasted_iota(jnp.int32, (NUM_ANCHORS, m_rows), 0)
    vals_rows, idx_rows = [], []
    for _ in range(K):
        m = jnp.max(c, axis=0, keepdims=True)
        idxk = jnp.min(jnp.where(c == m, iota_a, NUM_ANCHORS),
                       axis=0, keepdims=True)
        vals_rows.append(m)
        idx_rows.append(idxk.astype(jnp.float32))
        c = jnp.where(iota_a == idxk, -1.0, c)

    pad = jnp.zeros((NUM_ANCHORS - K, m_rows), jnp.float32)
    vals16 = jnp.concatenate(vals_rows + [pad], axis=0).T
    idx16 = jnp.concatenate(idx_rows + [pad], axis=0).T.astype(jnp.int32)

    anchor_of_lane = lax.broadcasted_iota(jnp.int32, (m_rows, NUM_ANCHORS * 4), 1) // 4
    det = jnp.zeros((m_rows, K * 4), jnp.float32)
    for k in range(K):
        valid = vals16[:, k : k + 1] > 0.5
        selw = jnp.where((anchor_of_lane == idx16[:, k : k + 1]) & valid, 1.0, 0.0)
        det = det + jnp.dot(bbox * selw,
                            e_ref[k * NUM_ANCHORS * 4 : (k + 1) * NUM_ANCHORS * 4, :],
                            preferred_element_type=jnp.float32)

    vals = vals16[:, :K]
    det_ref[:] = det
    vals_ref[:] = vals
    mask_ref[:] = vals > 0.5


def _placement_matrix() -> np.ndarray:
    e = np.zeros((K * NUM_ANCHORS * 4, K * 4), np.float32)
    for k in range(K):
        for j in range(NUM_ANCHORS * 4):
            e[k * NUM_ANCHORS * 4 + j, 4 * k + (j % 4)] = 1.0
    return e


@functools.partial(jax.jit, static_argnames=())
def kernel(features, bbox_W, bbox_b, conf_W, conf_b):
    B, T, F = features.shape
    R = B * T
    x = features.reshape(R, F)
    bb = bbox_b[None, :]
    cbT = conf_b[:, None]
    e = jnp.asarray(_placement_matrix())

    grid = (R // ROW_TILE,)
    det, vals, mask = pl.pallas_call(
        _head_kernel,
        grid=grid,
        in_specs=[
            pl.BlockSpec((ROW_TILE, F), lambda i: (i, 0)),
            pl.BlockSpec((F, NUM_ANCHORS * 4), lambda i: (0, 0)),
            pl.BlockSpec((1, NUM_ANCHORS * 4), lambda i: (0, 0)),
            pl.BlockSpec((F, NUM_ANCHORS), lambda i: (0, 0)),
            pl.BlockSpec((NUM_ANCHORS, 1), lambda i: (0, 0)),
            pl.BlockSpec((K * NUM_ANCHORS * 4, K * 4), lambda i: (0, 0)),
        ],
        out_specs=[
            pl.BlockSpec((ROW_TILE, K * 4), lambda i: (i, 0)),
            pl.BlockSpec((ROW_TILE, K), lambda i: (i, 0)),
            pl.BlockSpec((ROW_TILE, K), lambda i: (i, 0)),
        ],
        out_shape=[
            jax.ShapeDtypeStruct((R, K * 4), jnp.float32),
            jax.ShapeDtypeStruct((R, K), jnp.float32),
            jax.ShapeDtypeStruct((R, K), jnp.bool_),
        ],
        compiler_params=pltpu.CompilerParams(
            dimension_semantics=("parallel",),
        ),
    )(x, bbox_W, bb, conf_W, cbT, e)

    return (det.reshape(B, T, K, 4), vals.reshape(B, T, K), mask.reshape(B, T, K))
